# baseline (speedup 1.0000x reference)
import functools

import jax
import jax.numpy as jnp
import numpy as np
from jax import lax
from jax.experimental import pallas as pl
from jax.experimental.pallas import tpu as pltpu
from jax.experimental.pallas import tpu_sc as plsc

_N = 10000
_D = 128
_E = 160000
_H = 4
_DH = _D // _H
_BR = 1000
_NBLK = _N // _BR
_NT = 16
_EPT = _E // _NT
_C = 80
_NCH = _EPT // _C
_RPT = _N // _NT


def _build_consts():
    rep = np.zeros((_D, 4 * _D), np.float32)
    tile = np.zeros((_D, 4 * _D), np.float32)
    sum32 = np.zeros((4 * _D, 16), np.float32)
    bcast = np.zeros((16, 4 * _D), np.float32)
    red = np.zeros((4 * _D, _D), np.float32)
    scale = 1.0 / np.sqrt(_DH)
    for g in range(4):
        for h in range(4):
            for j in range(_DH):
                lane = g * _D + h * _DH + j
                rep[h * _DH + j, lane] = 1.0
                tile[g * _DH + j, lane] = 1.0
                sum32[lane, h * 4 + g] = scale
                bcast[h * 4 + g, lane] = 1.0
        for l in range(_D):
            red[g * _D + l, l] = 1.0
    pm = np.zeros((16, 48), np.float32)
    gm = np.zeros((16, 16), np.float32)
    for h in range(4):
        for g in range(4):
            for r in range(3):
                pm[h * 4 + g, 16 * r + h * 4 + ((g + r + 1) % 4)] = 1.0
            for g2 in range(4):
                gm[h * 4 + g, h * 4 + g2] = 1.0
    return rep, tile, sum32, pm, gm, bcast, red


def _attn_branch(xb, Aq, aq, Ak, ak, Av, av, sum32, pm, gm, bcast, red):
    f32 = jnp.float32
    dot = functools.partial(jnp.dot, preferred_element_type=f32)
    q4 = dot(xb, Aq) + aq
    k4 = dot(xb, Ak) + ak
    v4 = dot(xb, Av) + av
    L = dot(q4 * k4, sum32)
    rot = dot(L, pm)
    m = jnp.maximum(jnp.maximum(L, rot[:, 0:16]),
                    jnp.maximum(rot[:, 16:32], rot[:, 32:48]))
    E = jnp.exp(L - m)
    S = dot(E, gm)
    P4 = dot(E / S, bcast)
    return dot(P4 * v4, red)


def _dense_body(x_ref, Aqf, aqf, Akf, akf, Avf, avf,
                Aql, aql, Akl, akl, Avl, avl,
                sum32, pm, gm, bcast, red,
                of_ref, ol_ref):
    xb = x_ref[...]
    consts = (sum32[...], pm[...], gm[...], bcast[...], red[...])
    of_ref[...] = _attn_branch(xb, Aqf[...], aqf[...], Akf[...], akf[...],
                               Avf[...], avf[...], *consts)
    ol_ref[...] = _attn_branch(xb, Aql[...], aql[...], Akl[...], akl[...],
                               Avl[...], avl[...], *consts)


def _full(shape):
    return pl.BlockSpec(shape, lambda i: tuple(0 for _ in shape))


def _make_dense_call():
    rspec = pl.BlockSpec((_BR, _D), lambda i: (i, 0))
    in_specs = ([rspec]
                + [_full((_D, 4 * _D)), _full((1, 4 * _D))] * 6
                + [_full((4 * _D, 16)), _full((16, 48)), _full((16, 16)),
                   _full((16, 4 * _D)), _full((4 * _D, _D))])
    return pl.pallas_call(
        _dense_body,
        grid=(_NBLK,),
        in_specs=in_specs,
        out_specs=[rspec, rspec],
        out_shape=[jax.ShapeDtypeStruct((_N, _D), jnp.float32)] * 2,
    )


def _sc_body(src_f, dst_f, src_l, dst_l, feat_f, feat_l,
             ssum_f, ssum_l, cnt_f, cnt_l,
             srcv, dstv, r0, r1, ones, zb80, acc, csp,
             g0, g1, s0, s1, sem_c, sem_z):
    cid = lax.axis_index("c")
    tid = lax.axis_index("s")
    f32 = jnp.float32
    rows = [r0, r1]
    gsem = [g0, g1]
    ssem = [s0, s1]

    def _fill(i, carry):
        ones[pl.ds(i * 16, 16)] = jnp.ones((16,), f32)
        zb80[pl.ds(i * 16, 16)] = jnp.zeros((16,), f32)
        return carry

    lax.fori_loop(0, 5, _fill, 0)

    def _zrow(r, carry):
        def _zc(c2, carry2):
            r0[r, pl.ds(c2 * 16, 16)] = jnp.zeros((16,), f32)
            return carry2
        return lax.fori_loop(0, _D // 16, _zc, carry)

    lax.fori_loop(0, _C, _zrow, 0)

    def _zacc(k2, carry):
        pltpu.async_copy(r0, acc.at[pl.ds(tid * _RPT + k2 * _C, _C)], sem_z)
        return carry

    lax.fori_loop(0, 7, _zacc, 0)
    pltpu.async_copy(r0.at[pl.ds(0, 65)],
                     acc.at[pl.ds(tid * _RPT + 560, 65)], sem_z)

    def _zcnt(k2, carry):
        pltpu.async_copy(zb80, csp.at[pl.ds(tid * 640 + k2 * 80, 80)],
                         sem_c)
        return carry

    lax.fori_loop(0, 8, _zcnt, 0)

    def _zacc_d(k2, carry):
        pltpu.make_async_copy(r0, acc.at[pl.ds(tid * _RPT, _C)],
                              sem_z).wait()
        return carry

    lax.fori_loop(0, 7, _zacc_d, 0)
    pltpu.make_async_copy(r0.at[pl.ds(0, 65)],
                          acc.at[pl.ds(tid * _RPT + 560, 65)], sem_z).wait()

    def _zcnt_d(k2, carry):
        pltpu.make_async_copy(zb80, csp.at[pl.ds(tid * 640, 80)],
                              sem_c).wait()
        return carry

    lax.fori_loop(0, 8, _zcnt_d, 0)

    plsc.subcore_barrier()

    def _do(src_h, dst_h, feat_h, ssum_h, cnt_h):
        pltpu.async_copy(src_h.at[tid], srcv, g0)
        pltpu.async_copy(dst_h.at[tid], dstv, g1)
        pltpu.make_async_copy(src_h.at[tid], srcv, g0).wait()
        pltpu.make_async_copy(dst_h.at[tid], dstv, g1).wait()

        def g_start(j, r):
            pltpu.async_copy(feat_h.at[srcv.at[j]], rows[r], gsem[r])

        def g_wait(r):
            pltpu.make_async_copy(feat_h.at[srcv.at[0]], rows[r],
                                  gsem[r]).wait()

        def s_start(j, r):
            pltpu.async_copy(rows[r], acc.at[dstv.at[j]], ssem[r], add=True)
            pltpu.async_copy(ones, csp.at[dstv.at[j]], sem_c, add=True)

        def s_wait(r):
            pltpu.make_async_copy(rows[r], acc.at[dstv.at[0]],
                                  ssem[r]).wait()

        g_start(0, 0)
        g_start(1, 1)

        def _pair(i, carry):
            j = 2 * i
            g_wait(0)
            s_start(j, 0)
            g_wait(1)
            s_start(j + 1, 1)

            @pl.when(i < (_NCH - 1) // 2 - 1)
            def _():
                s_wait(0)
                g_start(j + 2, 0)
                s_wait(1)
                g_start(j + 3, 1)
            return carry

        lax.fori_loop(0, (_NCH - 1) // 2, _pair, 0)
        s_wait(0)
        g_start(_NCH - 1, 0)
        g_wait(0)
        s_start(_NCH - 1, 0)
        s_wait(1)
        s_wait(0)

        def _drain(i, carry):
            pltpu.make_async_copy(ones, csp.at[dstv.at[0]], sem_c).wait()
            return carry

        lax.fori_loop(0, _NCH, _drain, 0)

        plsc.subcore_barrier()
        pltpu.sync_copy(acc.at[pl.ds(tid * _RPT, _RPT)],
                        ssum_h.at[pl.ds(tid * _RPT, _RPT)])

        @pl.when(tid == 0)
        def _():
            pltpu.sync_copy(csp.at[pl.ds(0, _N)], cnt_h)

    @pl.when(cid == 0)
    def _():
        _do(src_f, dst_f, feat_f, ssum_f, cnt_f)

    @pl.when(cid == 1)
    def _():
        _do(src_l, dst_l, feat_l, ssum_l, cnt_l)


def _make_sc_call():
    mesh = plsc.VectorSubcoreMesh(core_axis_name="c", subcore_axis_name="s")
    return pl.kernel(
        _sc_body,
        out_type=[
            jax.ShapeDtypeStruct((_N, _D), jnp.float32),
            jax.ShapeDtypeStruct((_N, _D), jnp.float32),
            jax.ShapeDtypeStruct((_N,), jnp.float32),
            jax.ShapeDtypeStruct((_N,), jnp.float32),
        ],
        mesh=mesh,
        scratch_types=(
            [pltpu.VMEM((_NCH, _C), jnp.int32),
             pltpu.VMEM((_NCH, _C), jnp.int32)]
            + [pltpu.VMEM((_C, _D), jnp.float32)] * 2
            + [pltpu.VMEM((_C,), jnp.float32),
               pltpu.VMEM((80,), jnp.float32),
               pltpu.VMEM_SHARED((_N, _D), jnp.float32),
               pltpu.VMEM_SHARED((_NT * 640,), jnp.float32)]
            + [pltpu.SemaphoreType.DMA] * 6
        ),
        compiler_params=pltpu.CompilerParams(use_tc_tiling_on_sc=False),
    )


def _final_body(sf_ref, sl_ref, cf_ref, cl_ref, x_ref, g_ref, b_ref, o_ref):
    cf = jnp.maximum(cf_ref[...], 1.0)
    cl = jnp.maximum(cl_ref[...], 1.0)
    h = sf_ref[...] / cf + sl_ref[...] / cl + x_ref[...]
    mu = jnp.mean(h, axis=1, keepdims=True)
    d = h - mu
    var = jnp.mean(d * d, axis=1, keepdims=True)
    h = d * lax.rsqrt(var + 1e-5)
    o_ref[...] = h * g_ref[...] + b_ref[...]


def _make_final_call():
    rspec = pl.BlockSpec((_BR, _D), lambda i: (i, 0))
    cspec = pl.BlockSpec((_BR, 1), lambda i: (i, 0))
    vspec = pl.BlockSpec((1, _D), lambda i: (0, 0))
    return pl.pallas_call(
        _final_body,
        grid=(_NBLK,),
        in_specs=[rspec, rspec, cspec, cspec, rspec, vspec, vspec],
        out_specs=rspec,
        out_shape=jax.ShapeDtypeStruct((_N, _D), jnp.float32),
    )


@jax.jit
def kernel(x, edge_index_follows, edge_index_likes,
           W_follows, b_follows, Wq_follows, bq_follows, Wk_follows, bk_follows,
           Wv_follows, bv_follows,
           W_likes, b_likes, Wq_likes, bq_likes, Wk_likes, bk_likes,
           Wv_likes, bv_likes,
           ln_gamma, ln_beta):
    f32 = jnp.float32

    rep, tile, sum32, pm, gm, bcast, red = _build_consts()

    def prep(W, b, Wq, bq, Wk, bk, Wv, bv):
        out = []
        for Wp, bp, sel in ((Wq, bq, rep), (Wk, bk, tile), (Wv, bv, tile)):
            out.append(W.T @ Wp.T @ sel)
            out.append(((b @ Wp.T + bp).reshape(1, _D)) @ sel)
        return out

    wf = prep(W_follows, b_follows, Wq_follows, bq_follows,
              Wk_follows, bk_follows, Wv_follows, bv_follows)
    wl = prep(W_likes, b_likes, Wq_likes, bq_likes,
              Wk_likes, bk_likes, Wv_likes, bv_likes)
    consts = tuple(jnp.asarray(c) for c in (sum32, pm, gm, bcast, red))

    out_f, out_l = _make_dense_call()(x.astype(f32), *wf, *wl, *consts)

    eif = edge_index_follows.astype(jnp.int32)
    eil = edge_index_likes.astype(jnp.int32)
    src_f = eif[0].reshape(_NT, _NCH, _C)
    dst_f = eif[1].reshape(_NT, _NCH, _C)
    src_l = eil[0].reshape(_NT, _NCH, _C)
    dst_l = eil[1].reshape(_NT, _NCH, _C)

    ssum_f, ssum_l, cnt_f, cnt_l = _make_sc_call()(
        src_f, dst_f, src_l, dst_l, out_f, out_l)

    return _make_final_call()(ssum_f, ssum_l, cnt_f.reshape(_N, 1),
                              cnt_l.reshape(_N, 1), x.astype(f32),
                              ln_gamma.reshape(1, _D), ln_beta.reshape(1, _D))


# speedup vs baseline: 1.1010x; 1.1010x over previous
import functools

import jax
import jax.numpy as jnp
import numpy as np
from jax import lax
from jax.experimental import pallas as pl
from jax.experimental.pallas import tpu as pltpu
from jax.experimental.pallas import tpu_sc as plsc

_N = 10000
_D = 128
_E = 160000
_H = 4
_DH = _D // _H
_BR = 1000
_NBLK = _N // _BR
_NT = 16
_EPT = _E // _NT
_C = 40
_NCH = _EPT // _C
_NBUF = 5
_RPT = _N // _NT


def _build_consts():
    rep = np.zeros((_D, 4 * _D), np.float32)
    tile = np.zeros((_D, 4 * _D), np.float32)
    sum32 = np.zeros((4 * _D, 16), np.float32)
    bcast = np.zeros((16, 4 * _D), np.float32)
    red = np.zeros((4 * _D, _D), np.float32)
    scale = 1.0 / np.sqrt(_DH)
    for g in range(4):
        for h in range(4):
            for j in range(_DH):
                lane = g * _D + h * _DH + j
                rep[h * _DH + j, lane] = 1.0
                tile[g * _DH + j, lane] = 1.0
                sum32[lane, h * 4 + g] = scale
                bcast[h * 4 + g, lane] = 1.0
        for l in range(_D):
            red[g * _D + l, l] = 1.0
    pm = np.zeros((16, 48), np.float32)
    gm = np.zeros((16, 16), np.float32)
    for h in range(4):
        for g in range(4):
            for r in range(3):
                pm[h * 4 + g, 16 * r + h * 4 + ((g + r + 1) % 4)] = 1.0
            for g2 in range(4):
                gm[h * 4 + g, h * 4 + g2] = 1.0
    return rep, tile, sum32, pm, gm, bcast, red


def _attn_branch(xb, Aq, aq, Ak, ak, Av, av, sum32, pm, gm, bcast, red):
    f32 = jnp.float32
    dot = functools.partial(jnp.dot, preferred_element_type=f32)
    q4 = dot(xb, Aq) + aq
    k4 = dot(xb, Ak) + ak
    v4 = dot(xb, Av) + av
    L = dot(q4 * k4, sum32)
    rot = dot(L, pm)
    m = jnp.maximum(jnp.maximum(L, rot[:, 0:16]),
                    jnp.maximum(rot[:, 16:32], rot[:, 32:48]))
    E = jnp.exp(L - m)
    S = dot(E, gm)
    P4 = dot(E / S, bcast)
    return dot(P4 * v4, red)


def _dense_body(x_ref, Aqf, aqf, Akf, akf, Avf, avf,
                Aql, aql, Akl, akl, Avl, avl,
                sum32, pm, gm, bcast, red,
                of_ref, ol_ref):
    xb = x_ref[...]
    consts = (sum32[...], pm[...], gm[...], bcast[...], red[...])
    of_ref[...] = _attn_branch(xb, Aqf[...], aqf[...], Akf[...], akf[...],
                               Avf[...], avf[...], *consts)
    ol_ref[...] = _attn_branch(xb, Aql[...], aql[...], Akl[...], akl[...],
                               Avl[...], avl[...], *consts)


def _full(shape):
    return pl.BlockSpec(shape, lambda i: tuple(0 for _ in shape))


def _make_dense_call():
    rspec = pl.BlockSpec((_BR, _D), lambda i: (i, 0))
    in_specs = ([rspec]
                + [_full((_D, 4 * _D)), _full((1, 4 * _D))] * 6
                + [_full((4 * _D, 16)), _full((16, 48)), _full((16, 16)),
                   _full((16, 4 * _D)), _full((4 * _D, _D))])
    return pl.pallas_call(
        _dense_body,
        grid=(_NBLK,),
        in_specs=in_specs,
        out_specs=[rspec, rspec],
        out_shape=[jax.ShapeDtypeStruct((_N, _D), jnp.float32)] * 2,
    )


def _sc_body(src_f, dst_f, src_l, dst_l, feat_f, feat_l,
             ssum_f, ssum_l, cnt_f, cnt_l,
             srcv, dstv, r0, r1, r2, r3, r4, ones, zbuf, zb80, acc, csp,
             g0, g1, g2, g3, g4, s0, s1, s2, s3, s4, sem_c, sem_z):
    cid = lax.axis_index("c")
    tid = lax.axis_index("s")
    f32 = jnp.float32
    rows = [r0, r1, r2, r3, r4]
    gsem = [g0, g1, g2, g3, g4]
    ssem = [s0, s1, s2, s3, s4]

    def _fill(i, carry):
        ones[pl.ds(i * 16, 16)] = jnp.ones((16,), f32)
        zb80[pl.ds(i * 16, 16)] = jnp.zeros((16,), f32)
        return carry

    lax.fori_loop(0, 3, _fill, 0)

    def _fill2(i, carry):
        zb80[pl.ds(48 + i * 16, 16)] = jnp.zeros((16,), f32)
        return carry

    lax.fori_loop(0, 2, _fill2, 0)

    def _zrow(r, carry):
        def _zc(c2, carry2):
            zbuf[r, pl.ds(c2 * 16, 16)] = jnp.zeros((16,), f32)
            return carry2
        return lax.fori_loop(0, _D // 16, _zc, carry)

    lax.fori_loop(0, 25, _zrow, 0)

    def _zacc(k2, carry):
        pltpu.async_copy(zbuf, acc.at[pl.ds(tid * _RPT + k2 * 25, 25)],
                         sem_z)
        return carry

    lax.fori_loop(0, 25, _zacc, 0)

    def _zcnt(k2, carry):
        pltpu.async_copy(zb80, csp.at[pl.ds(tid * 640 + k2 * 80, 80)],
                         sem_c)
        return carry

    lax.fori_loop(0, 8, _zcnt, 0)

    def _zacc_d(k2, carry):
        pltpu.make_async_copy(zbuf, acc.at[pl.ds(tid * _RPT, 25)],
                              sem_z).wait()
        return carry

    lax.fori_loop(0, 25, _zacc_d, 0)

    def _zcnt_d(k2, carry):
        pltpu.make_async_copy(zb80, csp.at[pl.ds(tid * 640, 80)],
                              sem_c).wait()
        return carry

    lax.fori_loop(0, 8, _zcnt_d, 0)

    plsc.subcore_barrier()

    def _do(src_h, dst_h, feat_h, ssum_h, cnt_h):
        pltpu.async_copy(src_h.at[tid], srcv, g0)
        pltpu.async_copy(dst_h.at[tid], dstv, g1)
        pltpu.make_async_copy(src_h.at[tid], srcv, g0).wait()
        pltpu.make_async_copy(dst_h.at[tid], dstv, g1).wait()

        def g_start(j, r):
            pltpu.async_copy(feat_h.at[srcv.at[j]], rows[r], gsem[r])

        def g_wait(r):
            pltpu.make_async_copy(feat_h.at[srcv.at[0]], rows[r],
                                  gsem[r]).wait()

        def s_start(j, r):
            pltpu.async_copy(rows[r], acc.at[dstv.at[j]], ssem[r], add=True)
            pltpu.async_copy(ones.at[pl.ds(0, _C)], csp.at[dstv.at[j]],
                             sem_c, add=True)

        def s_wait(r):
            pltpu.make_async_copy(rows[r], acc.at[dstv.at[0]],
                                  ssem[r]).wait()

        for r in range(_NBUF):
            g_start(r, r)

        def _step(i, carry):
            for r in range(_NBUF):
                j = _NBUF * i + r
                g_wait(r)
                s_start(j, r)

            @pl.when(i < _NCH // _NBUF - 1)
            def _():
                for r in range(_NBUF):
                    s_wait(r)
                    g_start(_NBUF * (i + 1) + r, r)
            return carry

        lax.fori_loop(0, _NCH // _NBUF, _step, 0)
        for r in range(_NBUF):
            s_wait(r)

        def _drain(i, carry):
            pltpu.make_async_copy(ones.at[pl.ds(0, _C)], csp.at[dstv.at[0]],
                                  sem_c).wait()
            return carry

        lax.fori_loop(0, _NCH, _drain, 0)

        plsc.subcore_barrier()
        pltpu.sync_copy(acc.at[pl.ds(tid * _RPT, _RPT)],
                        ssum_h.at[pl.ds(tid * _RPT, _RPT)])

        @pl.when(tid == 0)
        def _():
            pltpu.sync_copy(csp.at[pl.ds(0, _N)], cnt_h)

    @pl.when(cid == 0)
    def _():
        _do(src_f, dst_f, feat_f, ssum_f, cnt_f)

    @pl.when(cid == 1)
    def _():
        _do(src_l, dst_l, feat_l, ssum_l, cnt_l)


def _make_sc_call():
    mesh = plsc.VectorSubcoreMesh(core_axis_name="c", subcore_axis_name="s")
    return pl.kernel(
        _sc_body,
        out_type=[
            jax.ShapeDtypeStruct((_N, _D), jnp.float32),
            jax.ShapeDtypeStruct((_N, _D), jnp.float32),
            jax.ShapeDtypeStruct((_N,), jnp.float32),
            jax.ShapeDtypeStruct((_N,), jnp.float32),
        ],
        mesh=mesh,
        scratch_types=(
            [pltpu.VMEM((_NCH, _C), jnp.int32),
             pltpu.VMEM((_NCH, _C), jnp.int32)]
            + [pltpu.VMEM((_C, _D), jnp.float32)] * _NBUF
            + [pltpu.VMEM((48,), jnp.float32),
               pltpu.VMEM((25, _D), jnp.float32),
               pltpu.VMEM((80,), jnp.float32),
               pltpu.VMEM_SHARED((_N, _D), jnp.float32),
               pltpu.VMEM_SHARED((_NT * 640,), jnp.float32)]
            + [pltpu.SemaphoreType.DMA] * 12
        ),
        compiler_params=pltpu.CompilerParams(use_tc_tiling_on_sc=False),
    )


def _final_body(sf_ref, sl_ref, cf_ref, cl_ref, x_ref, g_ref, b_ref, o_ref):
    cf = jnp.maximum(cf_ref[...], 1.0)
    cl = jnp.maximum(cl_ref[...], 1.0)
    h = sf_ref[...] / cf + sl_ref[...] / cl + x_ref[...]
    mu = jnp.mean(h, axis=1, keepdims=True)
    d = h - mu
    var = jnp.mean(d * d, axis=1, keepdims=True)
    h = d * lax.rsqrt(var + 1e-5)
    o_ref[...] = h * g_ref[...] + b_ref[...]


def _make_final_call():
    rspec = pl.BlockSpec((_BR, _D), lambda i: (i, 0))
    cspec = pl.BlockSpec((_BR, 1), lambda i: (i, 0))
    vspec = pl.BlockSpec((1, _D), lambda i: (0, 0))
    return pl.pallas_call(
        _final_body,
        grid=(_NBLK,),
        in_specs=[rspec, rspec, cspec, cspec, rspec, vspec, vspec],
        out_specs=rspec,
        out_shape=jax.ShapeDtypeStruct((_N, _D), jnp.float32),
    )


@jax.jit
def kernel(x, edge_index_follows, edge_index_likes,
           W_follows, b_follows, Wq_follows, bq_follows, Wk_follows, bk_follows,
           Wv_follows, bv_follows,
           W_likes, b_likes, Wq_likes, bq_likes, Wk_likes, bk_likes,
           Wv_likes, bv_likes,
           ln_gamma, ln_beta):
    f32 = jnp.float32

    rep, tile, sum32, pm, gm, bcast, red = _build_consts()

    def _expand_rep(B):
        return jnp.tile(B, (1, 4))

    def _expand_tile(B):
        r = B.shape[0]
        return jnp.broadcast_to(B.reshape(r, 4, 1, _DH),
                                (r, 4, 4, _DH)).reshape(r, 4 * _D)

    def prep(W, b, Wq, bq, Wk, bk, Wv, bv):
        Wb = jnp.concatenate([W.T, b.reshape(1, _D)], axis=0)
        out = []
        for Wp, bp, ex in ((Wq, bq, _expand_rep), (Wk, bk, _expand_tile),
                           (Wv, bv, _expand_tile)):
            C = jnp.dot(Wb, ex(Wp.T), preferred_element_type=f32)
            out.append(C[0:_D])
            out.append(C[_D:_D + 1] + ex(bp.reshape(1, _D)))
        return out

    wf = prep(W_follows, b_follows, Wq_follows, bq_follows,
              Wk_follows, bk_follows, Wv_follows, bv_follows)
    wl = prep(W_likes, b_likes, Wq_likes, bq_likes,
              Wk_likes, bk_likes, Wv_likes, bv_likes)
    consts = tuple(jnp.asarray(c) for c in (sum32, pm, gm, bcast, red))

    out_f, out_l = _make_dense_call()(x.astype(f32), *wf, *wl, *consts)

    eif = edge_index_follows.astype(jnp.int32)
    eil = edge_index_likes.astype(jnp.int32)
    src_f = eif[0].reshape(_NT, _NCH, _C)
    dst_f = eif[1].reshape(_NT, _NCH, _C)
    src_l = eil[0].reshape(_NT, _NCH, _C)
    dst_l = eil[1].reshape(_NT, _NCH, _C)

    ssum_f, ssum_l, cnt_f, cnt_l = _make_sc_call()(
        src_f, dst_f, src_l, dst_l, out_f, out_l)

    return _make_final_call()(ssum_f, ssum_l, cnt_f.reshape(_N, 1),
                              cnt_l.reshape(_N, 1), x.astype(f32),
                              ln_gamma.reshape(1, _D), ln_beta.reshape(1, _D))


# speedup vs baseline: 1.1350x; 1.0309x over previous
import functools

import jax
import jax.numpy as jnp
import numpy as np
from jax import lax
from jax.experimental import pallas as pl
from jax.experimental.pallas import tpu as pltpu
from jax.experimental.pallas import tpu_sc as plsc

_N = 10000
_D = 128
_E = 160000
_H = 4
_DH = _D // _H
_BR = 1000
_NBLK = _N // _BR
_NT = 16
_EPT = _E // _NT
_C = 40
_NCH = _EPT // _C
_NBUF = 5
_RPT = _N // _NT


def _build_consts():
    rep = np.zeros((_D, 4 * _D), np.float32)
    tile = np.zeros((_D, 4 * _D), np.float32)
    sum32 = np.zeros((4 * _D, 16), np.float32)
    bcast = np.zeros((16, 4 * _D), np.float32)
    red = np.zeros((4 * _D, _D), np.float32)
    scale = 1.0 / np.sqrt(_DH)
    for g in range(4):
        for h in range(4):
            for j in range(_DH):
                lane = g * _D + h * _DH + j
                rep[h * _DH + j, lane] = 1.0
                tile[g * _DH + j, lane] = 1.0
                sum32[lane, h * 4 + g] = scale
                bcast[h * 4 + g, lane] = 1.0
        for l in range(_D):
            red[g * _D + l, l] = 1.0
    pm = np.zeros((16, 48), np.float32)
    gm = np.zeros((16, 16), np.float32)
    for h in range(4):
        for g in range(4):
            for r in range(3):
                pm[h * 4 + g, 16 * r + h * 4 + ((g + r + 1) % 4)] = 1.0
            for g2 in range(4):
                gm[h * 4 + g, h * 4 + g2] = 1.0
    return rep, tile, sum32, pm, gm, bcast, red


def _attn_branch(xb, Aq, aq, Ak, ak, Av, av, sum32, pm, gm, bcast, red):
    f32 = jnp.float32
    dot = functools.partial(jnp.dot, preferred_element_type=f32)
    q4 = dot(xb, Aq) + aq
    k4 = dot(xb, Ak) + ak
    v4 = dot(xb, Av) + av
    L = dot(q4 * k4, sum32)
    rot = dot(L, pm)
    m = jnp.maximum(jnp.maximum(L, rot[:, 0:16]),
                    jnp.maximum(rot[:, 16:32], rot[:, 32:48]))
    E = jnp.exp(L - m)
    S = dot(E, gm)
    P4 = dot(E / S, bcast)
    return dot(P4 * v4, red)


def _dense_body(x_ref, Aqf, aqf, Akf, akf, Avf, avf,
                Aql, aql, Akl, akl, Avl, avl,
                sum32, pm, gm, bcast, red,
                of_ref, ol_ref):
    xb = x_ref[...]
    consts = (sum32[...], pm[...], gm[...], bcast[...], red[...])
    of_ref[...] = _attn_branch(xb, Aqf[...], aqf[...], Akf[...], akf[...],
                               Avf[...], avf[...], *consts)
    ol_ref[...] = _attn_branch(xb, Aql[...], aql[...], Akl[...], akl[...],
                               Avl[...], avl[...], *consts)


def _full(shape):
    return pl.BlockSpec(shape, lambda i: tuple(0 for _ in shape))


def _make_dense_call():
    rspec = pl.BlockSpec((_BR, _D), lambda i: (i, 0))
    in_specs = ([rspec]
                + [_full((_D, 4 * _D)), _full((1, 4 * _D))] * 6
                + [_full((4 * _D, 16)), _full((16, 48)), _full((16, 16)),
                   _full((16, 4 * _D)), _full((4 * _D, _D))])
    return pl.pallas_call(
        _dense_body,
        grid=(_NBLK,),
        in_specs=in_specs,
        out_specs=[rspec, rspec],
        out_shape=[jax.ShapeDtypeStruct((_N, _D), jnp.float32)] * 2,
    )


def _sc_body(src_f, dst_f, src_l, dst_l, feat_f, feat_l,
             ssum_f, ssum_l, cnt_f, cnt_l,
             srcv, dstv, r0, r1, r2, r3, r4, ones, zbuf, zb80, acc, csp,
             g0, g1, g2, g3, g4, s0, s1, s2, s3, s4, sem_c, sem_z):
    cid = lax.axis_index("c")
    tid = lax.axis_index("s")
    f32 = jnp.float32
    rows = [r0, r1, r2, r3, r4]
    gsem = [g0, g1, g2, g3, g4]
    ssem = [s0, s1, s2, s3, s4]

    def _fill(i, carry):
        ones[pl.ds(i * 16, 16)] = jnp.ones((16,), f32)
        zb80[pl.ds(i * 16, 16)] = jnp.zeros((16,), f32)
        return carry

    lax.fori_loop(0, 3, _fill, 0)

    def _fill2(i, carry):
        zb80[pl.ds(48 + i * 16, 16)] = jnp.zeros((16,), f32)
        return carry

    lax.fori_loop(0, 2, _fill2, 0)

    def _zrow(r, carry):
        def _zc(c2, carry2):
            zbuf[r, pl.ds(c2 * 16, 16)] = jnp.zeros((16,), f32)
            return carry2
        return lax.fori_loop(0, _D // 16, _zc, carry)

    lax.fori_loop(0, 25, _zrow, 0)

    def _zacc(k2, carry):
        pltpu.async_copy(zbuf, acc.at[pl.ds(tid * _RPT + k2 * 25, 25)],
                         sem_z)
        return carry

    lax.fori_loop(0, 25, _zacc, 0)

    def _zcnt(k2, carry):
        pltpu.async_copy(zb80, csp.at[pl.ds(tid * 640 + k2 * 80, 80)],
                         sem_c)
        return carry

    lax.fori_loop(0, 8, _zcnt, 0)

    def _zacc_d(k2, carry):
        pltpu.make_async_copy(zbuf, acc.at[pl.ds(tid * _RPT, 25)],
                              sem_z).wait()
        return carry

    lax.fori_loop(0, 25, _zacc_d, 0)

    def _zcnt_d(k2, carry):
        pltpu.make_async_copy(zb80, csp.at[pl.ds(tid * 640, 80)],
                              sem_c).wait()
        return carry

    lax.fori_loop(0, 8, _zcnt_d, 0)

    plsc.subcore_barrier()

    def _do(src_h, dst_h, feat_h, ssum_h, cnt_h):
        pltpu.async_copy(src_h.at[tid], srcv, g0)
        pltpu.async_copy(dst_h.at[tid], dstv, g1)
        pltpu.make_async_copy(src_h.at[tid], srcv, g0).wait()
        pltpu.make_async_copy(dst_h.at[tid], dstv, g1).wait()

        def g_start(j, r):
            pltpu.async_copy(feat_h.at[srcv.at[j]], rows[r], gsem[r])

        def g_wait(r):
            pltpu.make_async_copy(feat_h.at[srcv.at[0]], rows[r],
                                  gsem[r]).wait()

        def s_start(j, r):
            pltpu.async_copy(rows[r], acc.at[dstv.at[j]], ssem[r], add=True)
            pltpu.async_copy(ones.at[pl.ds(0, _C)], csp.at[dstv.at[j]],
                             sem_c, add=True)

        def s_wait(r):
            pltpu.make_async_copy(rows[r], acc.at[dstv.at[0]],
                                  ssem[r]).wait()

        for r in range(_NBUF):
            g_start(r, r)

        def _step(i, carry):
            for r in range(_NBUF):
                j = _NBUF * i + r
                g_wait(r)
                s_start(j, r)

            @pl.when(i < _NCH // _NBUF - 1)
            def _():
                for r in range(_NBUF):
                    s_wait(r)
                    g_start(_NBUF * (i + 1) + r, r)
            return carry

        lax.fori_loop(0, _NCH // _NBUF, _step, 0)
        for r in range(_NBUF):
            s_wait(r)

        def _drain(i, carry):
            pltpu.make_async_copy(ones.at[pl.ds(0, _C)], csp.at[dstv.at[0]],
                                  sem_c).wait()
            return carry

        lax.fori_loop(0, _NCH, _drain, 0)

        plsc.subcore_barrier()
        pltpu.sync_copy(acc.at[pl.ds(tid * _RPT, _RPT)],
                        ssum_h.at[pl.ds(tid * _RPT, _RPT)])

        @pl.when(tid == 0)
        def _():
            pltpu.sync_copy(csp.at[pl.ds(0, _N)], cnt_h)

    @pl.when(cid == 0)
    def _():
        _do(src_f, dst_f, feat_f, ssum_f, cnt_f)

    @pl.when(cid == 1)
    def _():
        _do(src_l, dst_l, feat_l, ssum_l, cnt_l)


def _make_sc_call():
    mesh = plsc.VectorSubcoreMesh(core_axis_name="c", subcore_axis_name="s")
    return pl.kernel(
        _sc_body,
        out_type=[
            jax.ShapeDtypeStruct((_N, _D), jnp.float32),
            jax.ShapeDtypeStruct((_N, _D), jnp.float32),
            jax.ShapeDtypeStruct((_N,), jnp.float32),
            jax.ShapeDtypeStruct((_N,), jnp.float32),
        ],
        mesh=mesh,
        scratch_types=(
            [pltpu.VMEM((_NCH, _C), jnp.int32),
             pltpu.VMEM((_NCH, _C), jnp.int32)]
            + [pltpu.VMEM((_C, _D), jnp.float32)] * _NBUF
            + [pltpu.VMEM((48,), jnp.float32),
               pltpu.VMEM((25, _D), jnp.float32),
               pltpu.VMEM((80,), jnp.float32),
               pltpu.VMEM_SHARED((_N, _D), jnp.float32),
               pltpu.VMEM_SHARED((_NT * 640,), jnp.float32)]
            + [pltpu.SemaphoreType.DMA] * 12
        ),
        compiler_params=pltpu.CompilerParams(use_tc_tiling_on_sc=False),
    )


def _final_body(sf_ref, sl_ref, cnt_ref, x_ref, g_ref, b_ref, o_ref):
    cf = jnp.maximum(cnt_ref[:, 0:1], 1.0)
    cl = jnp.maximum(cnt_ref[:, 1:2], 1.0)
    h = sf_ref[...] / cf + sl_ref[...] / cl + x_ref[...]
    mu = jnp.mean(h, axis=1, keepdims=True)
    d = h - mu
    var = jnp.mean(d * d, axis=1, keepdims=True)
    h = d * lax.rsqrt(var + 1e-5)
    o_ref[...] = h * g_ref[...] + b_ref[...]


def _make_final_call():
    rspec = pl.BlockSpec((_BR, _D), lambda i: (i, 0))
    cspec = pl.BlockSpec((_BR, 2), lambda i: (i, 0))
    vspec = pl.BlockSpec((1, _D), lambda i: (0, 0))
    return pl.pallas_call(
        _final_body,
        grid=(_NBLK,),
        in_specs=[rspec, rspec, cspec, rspec, vspec, vspec],
        out_specs=rspec,
        out_shape=jax.ShapeDtypeStruct((_N, _D), jnp.float32),
    )


@jax.jit
def kernel(x, edge_index_follows, edge_index_likes,
           W_follows, b_follows, Wq_follows, bq_follows, Wk_follows, bk_follows,
           Wv_follows, bv_follows,
           W_likes, b_likes, Wq_likes, bq_likes, Wk_likes, bk_likes,
           Wv_likes, bv_likes,
           ln_gamma, ln_beta):
    f32 = jnp.float32

    rep, tile, sum32, pm, gm, bcast, red = _build_consts()

    def _expand_rep(B):
        return jnp.tile(B, (1, 4))

    def _expand_tile(B):
        r = B.shape[0]
        return jnp.broadcast_to(B.reshape(r, 4, 1, _DH),
                                (r, 4, 4, _DH)).reshape(r, 4 * _D)

    def prep(W, b, Wq, bq, Wk, bk, Wv, bv):
        Wb = jnp.concatenate([W.T, b.reshape(1, _D)], axis=0)
        out = []
        for Wp, bp, ex in ((Wq, bq, _expand_rep), (Wk, bk, _expand_tile),
                           (Wv, bv, _expand_tile)):
            C = jnp.dot(Wb, ex(Wp.T), preferred_element_type=f32)
            out.append(C[0:_D])
            out.append(C[_D:_D + 1] + ex(bp.reshape(1, _D)))
        return out

    wf = prep(W_follows, b_follows, Wq_follows, bq_follows,
              Wk_follows, bk_follows, Wv_follows, bv_follows)
    wl = prep(W_likes, b_likes, Wq_likes, bq_likes,
              Wk_likes, bk_likes, Wv_likes, bv_likes)
    consts = tuple(jnp.asarray(c) for c in (sum32, pm, gm, bcast, red))

    out_f, out_l = _make_dense_call()(x.astype(f32), *wf, *wl, *consts)

    eif = edge_index_follows.astype(jnp.int32)
    eil = edge_index_likes.astype(jnp.int32)
    src_f = eif[0].reshape(_NT, _NCH, _C)
    dst_f = eif[1].reshape(_NT, _NCH, _C)
    src_l = eil[0].reshape(_NT, _NCH, _C)
    dst_l = eil[1].reshape(_NT, _NCH, _C)

    ssum_f, ssum_l, cnt_f, cnt_l = _make_sc_call()(
        src_f, dst_f, src_l, dst_l, out_f, out_l)

    cnt2 = jnp.stack([cnt_f, cnt_l], axis=1)

    return _make_final_call()(ssum_f, ssum_l, cnt2, x.astype(f32),
                              ln_gamma.reshape(1, _D), ln_beta.reshape(1, _D))


# speedup vs baseline: 1.2011x; 1.0582x over previous
import functools

import jax
import jax.numpy as jnp
import numpy as np
from jax import lax
from jax.experimental import pallas as pl
from jax.experimental.pallas import tpu as pltpu
from jax.experimental.pallas import tpu_sc as plsc

_N = 10000
_D = 128
_E = 160000
_H = 4
_DH = _D // _H
_BR = 1000
_NBLK = _N // _BR
_NT = 16
_EPT = _E // _NT
_C = 40
_NCH = _EPT // _C
_NBUF = 5
_RPT = _N // _NT


def _build_consts():
    rep = np.zeros((_D, 4 * _D), np.float32)
    tile = np.zeros((_D, 4 * _D), np.float32)
    sum32 = np.zeros((4 * _D, 16), np.float32)
    bcast = np.zeros((16, 4 * _D), np.float32)
    red = np.zeros((4 * _D, _D), np.float32)
    scale = 1.0 / np.sqrt(_DH)
    for g in range(4):
        for h in range(4):
            for j in range(_DH):
                lane = g * _D + h * _DH + j
                rep[h * _DH + j, lane] = 1.0
                tile[g * _DH + j, lane] = 1.0
                sum32[lane, h * 4 + g] = scale
                bcast[h * 4 + g, lane] = 1.0
        for l in range(_D):
            red[g * _D + l, l] = 1.0
    pm = np.zeros((16, 48), np.float32)
    gm = np.zeros((16, 16), np.float32)
    for h in range(4):
        for g in range(4):
            for r in range(3):
                pm[h * 4 + g, 16 * r + h * 4 + ((g + r + 1) % 4)] = 1.0
            for g2 in range(4):
                gm[h * 4 + g, h * 4 + g2] = 1.0
    return rep, tile, sum32, pm, gm, bcast, red


def _attn_branch(xb, Aq, aq, Ak, ak, Av, av, sum32, pm, gm, bcast, red):
    f32 = jnp.float32
    dot = functools.partial(jnp.dot, preferred_element_type=f32)
    q4 = dot(xb, Aq) + aq
    k4 = dot(xb, Ak) + ak
    v4 = dot(xb, Av) + av
    L = dot(q4 * k4, sum32)
    rot = dot(L, pm)
    m = jnp.maximum(jnp.maximum(L, rot[:, 0:16]),
                    jnp.maximum(rot[:, 16:32], rot[:, 32:48]))
    E = jnp.exp(L - m)
    S = dot(E, gm)
    P4 = dot(E / S, bcast)
    return dot(P4 * v4, red)


def _dense_body(x_ref, Aqf, aqf, Akf, akf, Avf, avf,
                Aql, aql, Akl, akl, Avl, avl,
                sum32, pm, gm, bcast, red,
                of_ref, ol_ref):
    xb = x_ref[...]
    consts = (sum32[...], pm[...], gm[...], bcast[...], red[...])
    of_ref[...] = _attn_branch(xb, Aqf[...], aqf[...], Akf[...], akf[...],
                               Avf[...], avf[...], *consts)
    ol_ref[...] = _attn_branch(xb, Aql[...], aql[...], Akl[...], akl[...],
                               Avl[...], avl[...], *consts)


def _full(shape):
    return pl.BlockSpec(shape, lambda i: tuple(0 for _ in shape))


def _ex_rep(M):
    return jnp.concatenate([M] * 4, axis=1)


def _ex_tile(M):
    cols = []
    for g in range(4):
        s = M[:, g * _DH:(g + 1) * _DH]
        cols.append(jnp.concatenate([s] * 4, axis=1))
    return jnp.concatenate(cols, axis=1)


def _prep_body(Wf, bf, Wqf, bqf, Wkf, bkf, Wvf, bvf,
               Wl, bl, Wql, bql, Wkl, bkl, Wvl, bvl,
               Aqf, aqf, Akf, akf, Avf, avf,
               Aql, aql, Akl, akl, Avl, avl):
    f32 = jnp.float32
    cdim0 = (((0,), (0,)), ((), ()))
    cdim1 = (((1,), (0,)), ((), ()))

    def one(W, b, Wp, bp, ex, A_ref, a_ref):
        E = ex(Wp[...].T)
        A_ref[...] = lax.dot_general(W[...], E, cdim0,
                                     preferred_element_type=f32)
        a_ref[...] = (lax.dot_general(b[...], E, cdim1,
                                      preferred_element_type=f32)
                      + ex(bp[...]))

    one(Wf, bf, Wqf, bqf, _ex_rep, Aqf, aqf)
    one(Wf, bf, Wkf, bkf, _ex_tile, Akf, akf)
    one(Wf, bf, Wvf, bvf, _ex_tile, Avf, avf)
    one(Wl, bl, Wql, bql, _ex_rep, Aql, aql)
    one(Wl, bl, Wkl, bkl, _ex_tile, Akl, akl)
    one(Wl, bl, Wvl, bvl, _ex_tile, Avl, avl)


def _make_prep_call():
    big = jax.ShapeDtypeStruct((_D, 4 * _D), jnp.float32)
    small = jax.ShapeDtypeStruct((1, 4 * _D), jnp.float32)
    return pl.pallas_call(
        _prep_body,
        out_shape=[big, small] * 6,
    )


def _make_dense_call():
    rspec = pl.BlockSpec((_BR, _D), lambda i: (i, 0))
    in_specs = ([rspec]
                + [_full((_D, 4 * _D)), _full((1, 4 * _D))] * 6
                + [_full((4 * _D, 16)), _full((16, 48)), _full((16, 16)),
                   _full((16, 4 * _D)), _full((4 * _D, _D))])
    return pl.pallas_call(
        _dense_body,
        grid=(_NBLK,),
        in_specs=in_specs,
        out_specs=[rspec, rspec],
        out_shape=[jax.ShapeDtypeStruct((_N, _D), jnp.float32)] * 2,
    )


def _sc_body(src_f, dst_f, src_l, dst_l, feat_f, feat_l,
             ssum_f, ssum_l, cnt_f, cnt_l,
             srcv, dstv, r0, r1, r2, r3, r4, ones, zbuf, zb80, acc, csp,
             g0, g1, g2, g3, g4, s0, s1, s2, s3, s4, sem_c, sem_z):
    cid = lax.axis_index("c")
    tid = lax.axis_index("s")
    f32 = jnp.float32
    rows = [r0, r1, r2, r3, r4]
    gsem = [g0, g1, g2, g3, g4]
    ssem = [s0, s1, s2, s3, s4]

    def _fill(i, carry):
        ones[pl.ds(i * 16, 16)] = jnp.ones((16,), f32)
        zb80[pl.ds(i * 16, 16)] = jnp.zeros((16,), f32)
        return carry

    lax.fori_loop(0, 3, _fill, 0)

    def _fill2(i, carry):
        zb80[pl.ds(48 + i * 16, 16)] = jnp.zeros((16,), f32)
        return carry

    lax.fori_loop(0, 2, _fill2, 0)

    def _zrow(r, carry):
        def _zc(c2, carry2):
            zbuf[r, pl.ds(c2 * 16, 16)] = jnp.zeros((16,), f32)
            return carry2
        return lax.fori_loop(0, _D // 16, _zc, carry)

    lax.fori_loop(0, 25, _zrow, 0)

    def _zacc(k2, carry):
        pltpu.async_copy(zbuf, acc.at[pl.ds(tid * _RPT + k2 * 25, 25)],
                         sem_z)
        return carry

    lax.fori_loop(0, 25, _zacc, 0)

    def _zcnt(k2, carry):
        pltpu.async_copy(zb80, csp.at[pl.ds(tid * 640 + k2 * 80, 80)],
                         sem_c)
        return carry

    lax.fori_loop(0, 8, _zcnt, 0)

    def _zacc_d(k2, carry):
        pltpu.make_async_copy(zbuf, acc.at[pl.ds(tid * _RPT, 25)],
                              sem_z).wait()
        return carry

    lax.fori_loop(0, 25, _zacc_d, 0)

    def _zcnt_d(k2, carry):
        pltpu.make_async_copy(zb80, csp.at[pl.ds(tid * 640, 80)],
                              sem_c).wait()
        return carry

    lax.fori_loop(0, 8, _zcnt_d, 0)

    plsc.subcore_barrier()

    def _do(src_h, dst_h, feat_h, ssum_h, cnt_h):
        pltpu.async_copy(src_h.at[tid], srcv, g0)
        pltpu.async_copy(dst_h.at[tid], dstv, g1)
        pltpu.make_async_copy(src_h.at[tid], srcv, g0).wait()
        pltpu.make_async_copy(dst_h.at[tid], dstv, g1).wait()

        def g_start(j, r):
            pltpu.async_copy(feat_h.at[srcv.at[j]], rows[r], gsem[r])

        def g_wait(r):
            pltpu.make_async_copy(feat_h.at[srcv.at[0]], rows[r],
                                  gsem[r]).wait()

        def s_start(j, r):
            pltpu.async_copy(rows[r], acc.at[dstv.at[j]], ssem[r], add=True)
            pltpu.async_copy(ones.at[pl.ds(0, _C)], csp.at[dstv.at[j]],
                             sem_c, add=True)

        def s_wait(r):
            pltpu.make_async_copy(rows[r], acc.at[dstv.at[0]],
                                  ssem[r]).wait()

        for r in range(_NBUF):
            g_start(r, r)

        def _step(i, carry):
            for r in range(_NBUF):
                j = _NBUF * i + r
                g_wait(r)
                s_start(j, r)

            @pl.when(i < _NCH // _NBUF - 1)
            def _():
                for r in range(_NBUF):
                    s_wait(r)
                    g_start(_NBUF * (i + 1) + r, r)
            return carry

        lax.fori_loop(0, _NCH // _NBUF, _step, 0)
        for r in range(_NBUF):
            s_wait(r)

        def _drain(i, carry):
            pltpu.make_async_copy(ones.at[pl.ds(0, _C)], csp.at[dstv.at[0]],
                                  sem_c).wait()
            return carry

        lax.fori_loop(0, _NCH, _drain, 0)

        plsc.subcore_barrier()
        pltpu.sync_copy(acc.at[pl.ds(tid * _RPT, _RPT)],
                        ssum_h.at[pl.ds(tid * _RPT, _RPT)])

        @pl.when(tid == 0)
        def _():
            pltpu.sync_copy(csp.at[pl.ds(0, _N)], cnt_h)

    @pl.when(cid == 0)
    def _():
        _do(src_f, dst_f, feat_f, ssum_f, cnt_f)

    @pl.when(cid == 1)
    def _():
        _do(src_l, dst_l, feat_l, ssum_l, cnt_l)


def _make_sc_call():
    mesh = plsc.VectorSubcoreMesh(core_axis_name="c", subcore_axis_name="s")
    return pl.kernel(
        _sc_body,
        out_type=[
            jax.ShapeDtypeStruct((_N, _D), jnp.float32),
            jax.ShapeDtypeStruct((_N, _D), jnp.float32),
            jax.ShapeDtypeStruct((_N,), jnp.float32),
            jax.ShapeDtypeStruct((_N,), jnp.float32),
        ],
        mesh=mesh,
        scratch_types=(
            [pltpu.VMEM((_NCH, _C), jnp.int32),
             pltpu.VMEM((_NCH, _C), jnp.int32)]
            + [pltpu.VMEM((_C, _D), jnp.float32)] * _NBUF
            + [pltpu.VMEM((48,), jnp.float32),
               pltpu.VMEM((25, _D), jnp.float32),
               pltpu.VMEM((80,), jnp.float32),
               pltpu.VMEM_SHARED((_N, _D), jnp.float32),
               pltpu.VMEM_SHARED((_NT * 640,), jnp.float32)]
            + [pltpu.SemaphoreType.DMA] * 12
        ),
        compiler_params=pltpu.CompilerParams(use_tc_tiling_on_sc=False),
    )


def _final_body(sf_ref, sl_ref, cnt_ref, x_ref, g_ref, b_ref, o_ref):
    cf = jnp.maximum(cnt_ref[:, 0:1], 1.0)
    cl = jnp.maximum(cnt_ref[:, 1:2], 1.0)
    h = sf_ref[...] / cf + sl_ref[...] / cl + x_ref[...]
    mu = jnp.mean(h, axis=1, keepdims=True)
    d = h - mu
    var = jnp.mean(d * d, axis=1, keepdims=True)
    h = d * lax.rsqrt(var + 1e-5)
    o_ref[...] = h * g_ref[...] + b_ref[...]


def _make_final_call():
    rspec = pl.BlockSpec((_BR, _D), lambda i: (i, 0))
    cspec = pl.BlockSpec((_BR, 2), lambda i: (i, 0))
    vspec = pl.BlockSpec((1, _D), lambda i: (0, 0))
    return pl.pallas_call(
        _final_body,
        grid=(_NBLK,),
        in_specs=[rspec, rspec, cspec, rspec, vspec, vspec],
        out_specs=rspec,
        out_shape=jax.ShapeDtypeStruct((_N, _D), jnp.float32),
    )


@jax.jit
def kernel(x, edge_index_follows, edge_index_likes,
           W_follows, b_follows, Wq_follows, bq_follows, Wk_follows, bk_follows,
           Wv_follows, bv_follows,
           W_likes, b_likes, Wq_likes, bq_likes, Wk_likes, bk_likes,
           Wv_likes, bv_likes,
           ln_gamma, ln_beta):
    f32 = jnp.float32

    rep, tile, sum32, pm, gm, bcast, red = _build_consts()

    folded = _make_prep_call()(
        W_follows, b_follows.reshape(1, _D), Wq_follows,
        bq_follows.reshape(1, _D), Wk_follows, bk_follows.reshape(1, _D),
        Wv_follows, bv_follows.reshape(1, _D),
        W_likes, b_likes.reshape(1, _D), Wq_likes,
        bq_likes.reshape(1, _D), Wk_likes, bk_likes.reshape(1, _D),
        Wv_likes, bv_likes.reshape(1, _D))
    consts = tuple(jnp.asarray(c) for c in (sum32, pm, gm, bcast, red))

    out_f, out_l = _make_dense_call()(x.astype(f32), *folded, *consts)

    eif = edge_index_follows.astype(jnp.int32)
    eil = edge_index_likes.astype(jnp.int32)
    src_f = eif[0].reshape(_NT, _NCH, _C)
    dst_f = eif[1].reshape(_NT, _NCH, _C)
    src_l = eil[0].reshape(_NT, _NCH, _C)
    dst_l = eil[1].reshape(_NT, _NCH, _C)

    ssum_f, ssum_l, cnt_f, cnt_l = _make_sc_call()(
        src_f, dst_f, src_l, dst_l, out_f, out_l)

    cnt2 = jnp.stack([cnt_f, cnt_l], axis=1)

    return _make_final_call()(ssum_f, ssum_l, cnt2, x.astype(f32),
                              ln_gamma.reshape(1, _D), ln_beta.reshape(1, _D))
